# Initial kernel scaffold; baseline (speedup 1.0000x reference)
import jax
import jax.numpy as jnp
from jax.experimental import pallas as pl


def kernel(input_tensor, W1, b1, W2, coords_table, rel_pos_index, local_window_size):
    raise NotImplementedError("write your pallas kernel here")


# speedup vs baseline: 4.0547x; 4.0547x over previous
import functools

import jax
import jax.numpy as jnp
from jax import lax
from jax.experimental import pallas as pl
from jax.experimental.pallas import tpu as pltpu
from jax.experimental.pallas import tpu_sc as plsc

NUM_HEADS = 16
SEQ = 512
NPOS = SEQ * SEQ
NTAB = 3375
NTAB_PAD = 3456
NC, NS = 2, 16
NW = NC * NS
PER_W = NPOS // NW
CHUNK = 2048
NCHUNK = PER_W // CHUNK
LANES = 16

HB = 4


def _mlp_body(coords_t_ref, w1t_ref, b1_ref, w2t_ref, out_ref):
    hid = jnp.dot(w1t_ref[...], coords_t_ref[...],
                  preferred_element_type=jnp.float32) + b1_ref[...]
    hid = jnp.maximum(hid, 0.0)
    logits = jnp.dot(w2t_ref[...], hid, preferred_element_type=jnp.float32)
    out_ref[...] = 16.0 / (1.0 + jnp.exp(-logits))


def _add_body(x_ref, b_ref, o_ref):
    o_ref[...] = x_ref[...] + b_ref[...][None]


def _sc_gather_t(table_t, idx_flat):
    mesh = plsc.VectorSubcoreMesh(core_axis_name="c", subcore_axis_name="s")

    @functools.partial(
        pl.kernel,
        out_type=jax.ShapeDtypeStruct((NUM_HEADS, NPOS), jnp.float32),
        mesh=mesh,
        compiler_params=pltpu.CompilerParams(needs_layout_passes=False),
        scratch_types=[
            pltpu.VMEM((NUM_HEADS * NTAB_PAD,), jnp.float32),
            pltpu.VMEM((CHUNK,), jnp.int32),
            pltpu.VMEM((NUM_HEADS, CHUNK), jnp.float32),
        ],
    )
    def k(tbl_hbm, idx_hbm, out_hbm, tbl_v, idx_v, out_v):
        wid = lax.axis_index("s") * NC + lax.axis_index("c")
        pltpu.sync_copy(tbl_hbm, tbl_v)
        for c in range(NCHUNK):
            base = wid * PER_W + c * CHUNK
            pltpu.sync_copy(idx_hbm.at[pl.ds(base, CHUNK)], idx_v)

            def body(i, _):
                p = i * LANES
                g = idx_v[pl.ds(p, LANES)]
                for h in range(NUM_HEADS):
                    gi = g + jnp.full((LANES,), h * NTAB_PAD, jnp.int32)
                    out_v[h, pl.ds(p, LANES)] = plsc.load_gather(tbl_v, [gi])
                return 0

            lax.fori_loop(0, CHUNK // LANES, body, 0)
            pltpu.sync_copy(out_v, out_hbm.at[:, pl.ds(base, CHUNK)])

    return k(table_t, idx_flat)


def kernel(input_tensor, W1, b1, W2, coords_table, rel_pos_index, local_window_size):
    coords_t = coords_table.reshape(-1, 3).astype(jnp.float32).T
    coords_tp = jnp.pad(coords_t, ((0, 5), (0, NTAB_PAD - NTAB)))
    w1t_p = jnp.pad(W1.astype(jnp.float32).T, ((0, 0), (0, 5)))

    table_t = pl.pallas_call(
        _mlp_body,
        out_shape=jax.ShapeDtypeStruct((NUM_HEADS, NTAB_PAD), jnp.float32),
    )(coords_tp, w1t_p, b1.reshape(-1, 1).astype(jnp.float32),
      W2.astype(jnp.float32).T)

    idx = rel_pos_index.reshape(-1).astype(jnp.int32)
    bias_hp = _sc_gather_t(table_t.reshape(-1), idx)
    bias3 = bias_hp.reshape(NUM_HEADS, SEQ, SEQ)

    nb, nh = input_tensor.shape[0], input_tensor.shape[1]
    out = pl.pallas_call(
        _add_body,
        grid=(nh // HB, nb),
        in_specs=[
            pl.BlockSpec((1, HB, SEQ, SEQ), lambda hb, b: (b, hb, 0, 0)),
            pl.BlockSpec((HB, SEQ, SEQ), lambda hb, b: (hb, 0, 0)),
        ],
        out_specs=pl.BlockSpec((1, HB, SEQ, SEQ), lambda hb, b: (b, hb, 0, 0)),
        out_shape=jax.ShapeDtypeStruct(input_tensor.shape, input_tensor.dtype),
    )(input_tensor, bias3)
    return out


# speedup vs baseline: 4.0968x; 1.0104x over previous
import functools

import jax
import jax.numpy as jnp
from jax import lax
from jax.experimental import pallas as pl
from jax.experimental.pallas import tpu as pltpu
from jax.experimental.pallas import tpu_sc as plsc

NUM_HEADS = 16
SEQ = 512
NPOS = SEQ * SEQ
NTAB = 3375
NTAB_PAD = 3456
NC, NS = 2, 16
NW = NC * NS
PER_W = NPOS // NW
CHUNK = 2048
NCHUNK = PER_W // CHUNK
LANES = 16

HB = 8


def _mlp_body(coords_t_ref, w1t_ref, b1_ref, w2t_ref, out_ref):
    hid = jnp.dot(w1t_ref[...], coords_t_ref[...],
                  preferred_element_type=jnp.float32) + b1_ref[...]
    hid = jnp.maximum(hid, 0.0)
    logits = jnp.dot(w2t_ref[...], hid, preferred_element_type=jnp.float32)
    out_ref[...] = 16.0 / (1.0 + jnp.exp(-logits))


def _add_body(x_ref, b_ref, o_ref):
    o_ref[...] = x_ref[...] + b_ref[...][None]


def _sc_gather_t(table_t, idx_flat):
    mesh = plsc.VectorSubcoreMesh(core_axis_name="c", subcore_axis_name="s")

    @functools.partial(
        pl.kernel,
        out_type=jax.ShapeDtypeStruct((NUM_HEADS, NPOS), jnp.float32),
        mesh=mesh,
        compiler_params=pltpu.CompilerParams(needs_layout_passes=False),
        scratch_types=[
            pltpu.VMEM((NUM_HEADS * NTAB_PAD,), jnp.float32),
            pltpu.VMEM((CHUNK,), jnp.int32),
            pltpu.VMEM((NUM_HEADS, CHUNK), jnp.float32),
        ],
    )
    def k(tbl_hbm, idx_hbm, out_hbm, tbl_v, idx_v, out_v):
        wid = lax.axis_index("s") * NC + lax.axis_index("c")
        pltpu.sync_copy(tbl_hbm, tbl_v)
        for c in range(NCHUNK):
            base = wid * PER_W + c * CHUNK
            pltpu.sync_copy(idx_hbm.at[pl.ds(base, CHUNK)], idx_v)

            def body(i, _):
                p = i * LANES
                g = idx_v[pl.ds(p, LANES)]
                for h in range(NUM_HEADS):
                    gi = g + jnp.full((LANES,), h * NTAB_PAD, jnp.int32)
                    out_v[h, pl.ds(p, LANES)] = plsc.load_gather(tbl_v, [gi])
                return 0

            lax.fori_loop(0, CHUNK // LANES, body, 0)
            pltpu.sync_copy(out_v, out_hbm.at[:, pl.ds(base, CHUNK)])

    return k(table_t, idx_flat)


def kernel(input_tensor, W1, b1, W2, coords_table, rel_pos_index, local_window_size):
    coords_t = coords_table.reshape(-1, 3).astype(jnp.float32).T
    coords_tp = jnp.pad(coords_t, ((0, 5), (0, NTAB_PAD - NTAB)))
    w1t_p = jnp.pad(W1.astype(jnp.float32).T, ((0, 0), (0, 5)))

    table_t = pl.pallas_call(
        _mlp_body,
        out_shape=jax.ShapeDtypeStruct((NUM_HEADS, NTAB_PAD), jnp.float32),
    )(coords_tp, w1t_p, b1.reshape(-1, 1).astype(jnp.float32),
      W2.astype(jnp.float32).T)

    idx = rel_pos_index.reshape(-1).astype(jnp.int32)
    bias_hp = _sc_gather_t(table_t.reshape(-1), idx)
    bias3 = bias_hp.reshape(NUM_HEADS, SEQ, SEQ)

    nb, nh = input_tensor.shape[0], input_tensor.shape[1]
    out = pl.pallas_call(
        _add_body,
        grid=(nh // HB, nb),
        in_specs=[
            pl.BlockSpec((1, HB, SEQ, SEQ), lambda hb, b: (b, hb, 0, 0)),
            pl.BlockSpec((HB, SEQ, SEQ), lambda hb, b: (hb, 0, 0)),
        ],
        out_specs=pl.BlockSpec((1, HB, SEQ, SEQ), lambda hb, b: (b, hb, 0, 0)),
        out_shape=jax.ShapeDtypeStruct(input_tensor.shape, input_tensor.dtype),
    )(input_tensor, bias3)
    return out


# speedup vs baseline: 4.6727x; 1.1406x over previous
import functools

import jax
import jax.numpy as jnp
from jax import lax
from jax.experimental import pallas as pl
from jax.experimental.pallas import tpu as pltpu
from jax.experimental.pallas import tpu_sc as plsc

NUM_HEADS = 16
SEQ = 512
NPOS = SEQ * SEQ
NTAB = 3375
NTAB_PAD = 3456
NC, NS = 2, 16
NW = NC * NS
PER_W = NPOS // NW
CHUNK = 2048
NCHUNK = PER_W // CHUNK
LANES = 16

HB = 8


def _mlp_body(coords_t_ref, w1t_ref, b1_ref, w2t_ref, out_ref):
    hid = jnp.dot(w1t_ref[...], coords_t_ref[...],
                  preferred_element_type=jnp.float32) + b1_ref[...]
    hid = jnp.maximum(hid, 0.0)
    logits = jnp.dot(w2t_ref[...], hid, preferred_element_type=jnp.float32)
    out_ref[...] = 16.0 / (1.0 + jnp.exp(-logits))


def _add_body(x_ref, b_ref, o_ref):
    o_ref[...] = x_ref[...] + b_ref[...][None]


def _sc_gather_t(table_t, idx_flat):
    mesh = plsc.VectorSubcoreMesh(core_axis_name="c", subcore_axis_name="s")

    @functools.partial(
        pl.kernel,
        out_type=jax.ShapeDtypeStruct((NUM_HEADS, NPOS), jnp.float32),
        mesh=mesh,
        compiler_params=pltpu.CompilerParams(needs_layout_passes=False),
        scratch_types=[
            pltpu.VMEM((NUM_HEADS * NTAB_PAD,), jnp.float32),
            pltpu.VMEM((PER_W,), jnp.int32),
            pltpu.VMEM((NUM_HEADS, CHUNK), jnp.float32),
            pltpu.VMEM((NUM_HEADS, CHUNK), jnp.float32),
            pltpu.SemaphoreType.DMA,
            pltpu.SemaphoreType.DMA,
        ],
    )
    def k(tbl_hbm, idx_hbm, out_hbm, tbl_v, idx_v, out_v0, out_v1, sem0, sem1):
        wid = lax.axis_index("s") * NC + lax.axis_index("c")
        base = wid * PER_W
        pltpu.sync_copy(tbl_hbm, tbl_v)
        pltpu.sync_copy(idx_hbm.at[pl.ds(base, PER_W)], idx_v)
        bufs = (out_v0, out_v1)
        sems = (sem0, sem1)
        copies = [None, None]
        for c in range(NCHUNK):
            out_v = bufs[c % 2]
            if copies[c % 2] is not None:
                copies[c % 2].wait()

            @plsc.parallel_loop(0, CHUNK, LANES, unroll=2)
            def body(p, out_v=out_v, off=c * CHUNK):
                g = idx_v[pl.ds(off + p, LANES)]
                for h in range(NUM_HEADS):
                    out_v[h, pl.ds(p, LANES)] = plsc.load_gather(
                        tbl_v, [g + h * NTAB_PAD])

            copies[c % 2] = pltpu.async_copy(
                out_v, out_hbm.at[:, pl.ds(base + c * CHUNK, CHUNK)],
                sems[c % 2])
        for cp in copies:
            if cp is not None:
                cp.wait()

    return k(table_t, idx_flat)


def kernel(input_tensor, W1, b1, W2, coords_table, rel_pos_index, local_window_size):
    coords_t = coords_table.reshape(-1, 3).astype(jnp.float32).T
    coords_tp = jnp.pad(coords_t, ((0, 5), (0, NTAB_PAD - NTAB)))
    w1t_p = jnp.pad(W1.astype(jnp.float32).T, ((0, 0), (0, 5)))

    table_t = pl.pallas_call(
        _mlp_body,
        out_shape=jax.ShapeDtypeStruct((NUM_HEADS, NTAB_PAD), jnp.float32),
    )(coords_tp, w1t_p, b1.reshape(-1, 1).astype(jnp.float32),
      W2.astype(jnp.float32).T)

    idx = rel_pos_index.reshape(-1).astype(jnp.int32)
    bias_hp = _sc_gather_t(table_t.reshape(-1), idx)
    bias3 = bias_hp.reshape(NUM_HEADS, SEQ, SEQ)

    nb, nh = input_tensor.shape[0], input_tensor.shape[1]
    out = pl.pallas_call(
        _add_body,
        grid=(nh // HB, nb),
        in_specs=[
            pl.BlockSpec((1, HB, SEQ, SEQ), lambda hb, b: (b, hb, 0, 0)),
            pl.BlockSpec((HB, SEQ, SEQ), lambda hb, b: (hb, 0, 0)),
        ],
        out_specs=pl.BlockSpec((1, HB, SEQ, SEQ), lambda hb, b: (b, hb, 0, 0)),
        out_shape=jax.ShapeDtypeStruct(input_tensor.shape, input_tensor.dtype),
    )(input_tensor, bias3)
    return out


# speedup vs baseline: 5.0469x; 1.0801x over previous
import functools

import jax
import jax.numpy as jnp
from jax import lax
from jax.experimental import pallas as pl
from jax.experimental.pallas import tpu as pltpu
from jax.experimental.pallas import tpu_sc as plsc

NUM_HEADS = 16
SEQ = 512
NPOS = SEQ * SEQ
NTAB = 3375
NTAB_PAD = 3456
NC, NS = 2, 16
NW = NC * NS
PER_W = NPOS // NW
ROWS_W = PER_W // SEQ
CHUNK = 4096
CROWS = CHUNK // SEQ
NCHUNK = PER_W // CHUNK
LANES = 16

HB = 8


def _mlp_body(coords_t_ref, w1t_ref, b1_ref, w2t_ref, out_ref):
    hid = jnp.dot(w1t_ref[...], coords_t_ref[...],
                  preferred_element_type=jnp.float32) + b1_ref[...]
    hid = jnp.maximum(hid, 0.0)
    logits = jnp.dot(w2t_ref[...], hid, preferred_element_type=jnp.float32)
    out_ref[...] = 16.0 / (1.0 + jnp.exp(-logits))


def _add_body(x_ref, b_ref, o_ref):
    o_ref[...] = x_ref[...] + b_ref[...][None]


def _sc_gather_t(table_t, idx_flat):
    mesh = plsc.VectorSubcoreMesh(core_axis_name="c", subcore_axis_name="s")

    @functools.partial(
        pl.kernel,
        out_type=jax.ShapeDtypeStruct((NUM_HEADS, SEQ, SEQ), jnp.float32),
        mesh=mesh,
        compiler_params=pltpu.CompilerParams(needs_layout_passes=False),
        scratch_types=[
            pltpu.VMEM((NUM_HEADS * NTAB_PAD,), jnp.float32),
            pltpu.VMEM((ROWS_W, SEQ), jnp.int32),
            pltpu.VMEM((NUM_HEADS, CROWS, SEQ), jnp.float32),
            pltpu.SemaphoreType.DMA,
        ],
    )
    def k(tbl_hbm, idx_hbm, out_hbm, tbl_v, idx_v, out_v, sem):
        wid = lax.axis_index("s") * NC + lax.axis_index("c")
        row0 = wid * ROWS_W
        pltpu.sync_copy(tbl_hbm, tbl_v)
        pltpu.sync_copy(idx_hbm.at[pl.ds(row0, ROWS_W), :], idx_v)
        cp = None
        for c in range(NCHUNK):
            if cp is not None:
                cp.wait()

            @plsc.parallel_loop(0, CHUNK, LANES, unroll=2)
            def body(p, off=c * CHUNK):
                q = off + p
                g = idx_v[q // SEQ, pl.ds(q % SEQ, LANES)]
                for h in range(NUM_HEADS):
                    out_v[h, p // SEQ, pl.ds(p % SEQ, LANES)] = plsc.load_gather(
                        tbl_v, [g + h * NTAB_PAD])

            cp = pltpu.async_copy(
                out_v, out_hbm.at[:, pl.ds(row0 + c * CROWS, CROWS), :], sem)
        cp.wait()

    return k(table_t, idx_flat)


def kernel(input_tensor, W1, b1, W2, coords_table, rel_pos_index, local_window_size):
    coords_t = coords_table.reshape(-1, 3).astype(jnp.float32).T
    coords_tp = jnp.pad(coords_t, ((0, 5), (0, NTAB_PAD - NTAB)))
    w1t_p = jnp.pad(W1.astype(jnp.float32).T, ((0, 0), (0, 5)))

    table_t = pl.pallas_call(
        _mlp_body,
        out_shape=jax.ShapeDtypeStruct((NUM_HEADS, NTAB_PAD), jnp.float32),
    )(coords_tp, w1t_p, b1.reshape(-1, 1).astype(jnp.float32),
      W2.astype(jnp.float32).T)

    idx = rel_pos_index.astype(jnp.int32)
    bias3 = _sc_gather_t(table_t.reshape(-1), idx)

    nb, nh = input_tensor.shape[0], input_tensor.shape[1]
    out = pl.pallas_call(
        _add_body,
        grid=(nh // HB, nb),
        in_specs=[
            pl.BlockSpec((1, HB, SEQ, SEQ), lambda hb, b: (b, hb, 0, 0)),
            pl.BlockSpec((HB, SEQ, SEQ), lambda hb, b: (hb, 0, 0)),
        ],
        out_specs=pl.BlockSpec((1, HB, SEQ, SEQ), lambda hb, b: (b, hb, 0, 0)),
        out_shape=jax.ShapeDtypeStruct(input_tensor.shape, input_tensor.dtype),
    )(input_tensor, bias3)
    return out


# speedup vs baseline: 5.1758x; 1.0255x over previous
import functools

import jax
import jax.numpy as jnp
from jax import lax
from jax.experimental import pallas as pl
from jax.experimental.pallas import tpu as pltpu
from jax.experimental.pallas import tpu_sc as plsc

NUM_HEADS = 16
SEQ = 512
NPOS = SEQ * SEQ
NTAB = 3375
NTAB_PAD = 3456
NC, NS = 2, 16
NW = NC * NS
HSPLIT = 2
HW = NUM_HEADS // HSPLIT
ROWS_W = SEQ // (NW // HSPLIT)
CROWS = 8
CHUNK = CROWS * SEQ
NCHUNK = ROWS_W // CROWS
LANES = 16

HB = 8


def _mlp_body(coords_t_ref, w1t_ref, b1_ref, w2t_ref, out_ref):
    hid = jnp.dot(w1t_ref[...], coords_t_ref[...],
                  preferred_element_type=jnp.float32) + b1_ref[...]
    hid = jnp.maximum(hid, 0.0)
    logits = jnp.dot(w2t_ref[...], hid, preferred_element_type=jnp.float32)
    out_ref[...] = 16.0 / (1.0 + jnp.exp(-logits))


def _add_body(x_ref, b_ref, o_ref):
    o_ref[...] = x_ref[...] + b_ref[...][None]


def _sc_gather_t(table_t, idx_flat):
    mesh = plsc.VectorSubcoreMesh(core_axis_name="c", subcore_axis_name="s")

    @functools.partial(
        pl.kernel,
        out_type=jax.ShapeDtypeStruct((NUM_HEADS, SEQ, SEQ), jnp.float32),
        mesh=mesh,
        compiler_params=pltpu.CompilerParams(needs_layout_passes=False),
        scratch_types=[
            pltpu.VMEM((HW * NTAB_PAD,), jnp.float32),
            pltpu.VMEM((ROWS_W, SEQ), jnp.int32),
            pltpu.VMEM((HW, CROWS, SEQ), jnp.float32),
            pltpu.VMEM((HW, CROWS, SEQ), jnp.float32),
            pltpu.SemaphoreType.DMA,
            pltpu.SemaphoreType.DMA,
        ],
    )
    def k(tbl_hbm, idx_hbm, out_hbm, tbl_v, idx_v, out_v0, out_v1, sem0, sem1):
        wid = lax.axis_index("s") * NC + lax.axis_index("c")
        h0 = (wid // (NW // HSPLIT)) * HW
        row0 = (wid % (NW // HSPLIT)) * ROWS_W
        pltpu.sync_copy(tbl_hbm.at[pl.ds(h0 * NTAB_PAD, HW * NTAB_PAD)], tbl_v)
        pltpu.sync_copy(idx_hbm.at[pl.ds(row0, ROWS_W), :], idx_v)
        bufs = (out_v0, out_v1)
        sems = (sem0, sem1)
        copies = [None, None]
        for c in range(NCHUNK):
            out_v = bufs[c % 2]
            if copies[c % 2] is not None:
                copies[c % 2].wait()

            @plsc.parallel_loop(0, CHUNK, LANES, unroll=2)
            def body(p, out_v=out_v, off=c * CHUNK):
                q = off + p
                g = idx_v[q // SEQ, pl.ds(q % SEQ, LANES)]
                for h in range(HW):
                    out_v[h, p // SEQ, pl.ds(p % SEQ, LANES)] = plsc.load_gather(
                        tbl_v, [g + h * NTAB_PAD])

            copies[c % 2] = pltpu.async_copy(
                out_v,
                out_hbm.at[pl.ds(h0, HW), pl.ds(row0 + c * CROWS, CROWS), :],
                sems[c % 2])
        for cp in copies:
            if cp is not None:
                cp.wait()

    return k(table_t, idx_flat)


def kernel(input_tensor, W1, b1, W2, coords_table, rel_pos_index, local_window_size):
    coords_t = coords_table.reshape(-1, 3).astype(jnp.float32).T
    coords_tp = jnp.pad(coords_t, ((0, 5), (0, NTAB_PAD - NTAB)))
    w1t_p = jnp.pad(W1.astype(jnp.float32).T, ((0, 0), (0, 5)))

    table_t = pl.pallas_call(
        _mlp_body,
        out_shape=jax.ShapeDtypeStruct((NUM_HEADS, NTAB_PAD), jnp.float32),
    )(coords_tp, w1t_p, b1.reshape(-1, 1).astype(jnp.float32),
      W2.astype(jnp.float32).T)

    idx = rel_pos_index.astype(jnp.int32)
    bias3 = _sc_gather_t(table_t.reshape(-1), idx)

    nb, nh = input_tensor.shape[0], input_tensor.shape[1]
    out = pl.pallas_call(
        _add_body,
        grid=(nh // HB, nb),
        in_specs=[
            pl.BlockSpec((1, HB, SEQ, SEQ), lambda hb, b: (b, hb, 0, 0)),
            pl.BlockSpec((HB, SEQ, SEQ), lambda hb, b: (hb, 0, 0)),
        ],
        out_specs=pl.BlockSpec((1, HB, SEQ, SEQ), lambda hb, b: (b, hb, 0, 0)),
        out_shape=jax.ShapeDtypeStruct(input_tensor.shape, input_tensor.dtype),
    )(input_tensor, bias3)
    return out


# speedup vs baseline: 5.1773x; 1.0003x over previous
import functools

import jax
import jax.numpy as jnp
from jax import lax
from jax.experimental import pallas as pl
from jax.experimental.pallas import tpu as pltpu
from jax.experimental.pallas import tpu_sc as plsc

NUM_HEADS = 16
SEQ = 512
NPOS = SEQ * SEQ
NTAB = 3375
NTAB_PAD = 3456
NC, NS = 2, 16
NW = NC * NS
HSPLIT = 2
HW = NUM_HEADS // HSPLIT
ROWS_W = SEQ // (NW // HSPLIT)
CROWS = 8
CHUNK = CROWS * SEQ
NCHUNK = ROWS_W // CROWS
LANES = 16

HB = 8


def _mlp_body(coords_t_ref, w1t_ref, b1_ref, w2t_ref, out_ref):
    hid = jnp.dot(w1t_ref[...], coords_t_ref[...],
                  preferred_element_type=jnp.float32) + b1_ref[...]
    hid = jnp.maximum(hid, 0.0)
    logits = jnp.dot(w2t_ref[...], hid, preferred_element_type=jnp.float32)
    out_ref[...] = 16.0 / (1.0 + jnp.exp(-logits))


def _add_body(x_ref, b_ref, o_ref):
    o_ref[...] = x_ref[...] + b_ref[...][None]


def _sc_gather_t(table_t, idx_flat):
    mesh = plsc.VectorSubcoreMesh(core_axis_name="c", subcore_axis_name="s")

    @functools.partial(
        pl.kernel,
        out_type=jax.ShapeDtypeStruct((NUM_HEADS, SEQ, SEQ), jnp.float32),
        mesh=mesh,
        compiler_params=pltpu.CompilerParams(needs_layout_passes=False),
        scratch_types=[
            pltpu.VMEM((HW * NTAB_PAD,), jnp.float32),
            pltpu.VMEM((ROWS_W, SEQ), jnp.int32),
            pltpu.VMEM((HW, CROWS, SEQ), jnp.float32),
            pltpu.VMEM((HW, CROWS, SEQ), jnp.float32),
            pltpu.SemaphoreType.DMA,
            pltpu.SemaphoreType.DMA,
        ],
    )
    def k(tbl_hbm, idx_hbm, out_hbm, tbl_v, idx_v, out_v0, out_v1, sem0, sem1):
        wid = lax.axis_index("s") * NC + lax.axis_index("c")
        h0 = (wid // (NW // HSPLIT)) * HW
        row0 = (wid % (NW // HSPLIT)) * ROWS_W
        pltpu.sync_copy(tbl_hbm.at[pl.ds(h0 * NTAB_PAD, HW * NTAB_PAD)], tbl_v)
        pltpu.sync_copy(idx_hbm.at[pl.ds(row0, ROWS_W), :], idx_v)
        bufs = (out_v0, out_v1)
        sems = (sem0, sem1)
        copies = [None, None]
        for c in range(NCHUNK):
            out_v = bufs[c % 2]
            if copies[c % 2] is not None:
                copies[c % 2].wait()

            @plsc.parallel_loop(0, CHUNK, LANES, unroll=4)
            def body(p, out_v=out_v, off=c * CHUNK):
                q = off + p
                g = idx_v[q // SEQ, pl.ds(q % SEQ, LANES)]
                for h in range(HW):
                    out_v[h, p // SEQ, pl.ds(p % SEQ, LANES)] = plsc.load_gather(
                        tbl_v, [g + h * NTAB_PAD])

            copies[c % 2] = pltpu.async_copy(
                out_v,
                out_hbm.at[pl.ds(h0, HW), pl.ds(row0 + c * CROWS, CROWS), :],
                sems[c % 2])
        for cp in copies:
            if cp is not None:
                cp.wait()

    return k(table_t, idx_flat)


def kernel(input_tensor, W1, b1, W2, coords_table, rel_pos_index, local_window_size):
    coords_t = coords_table.reshape(-1, 3).astype(jnp.float32).T
    coords_tp = jnp.pad(coords_t, ((0, 5), (0, NTAB_PAD - NTAB)))
    w1t_p = jnp.pad(W1.astype(jnp.float32).T, ((0, 0), (0, 5)))

    table_t = pl.pallas_call(
        _mlp_body,
        out_shape=jax.ShapeDtypeStruct((NUM_HEADS, NTAB_PAD), jnp.float32),
    )(coords_tp, w1t_p, b1.reshape(-1, 1).astype(jnp.float32),
      W2.astype(jnp.float32).T)

    idx = rel_pos_index.astype(jnp.int32)
    bias3 = _sc_gather_t(table_t.reshape(-1), idx)

    nb, nh = input_tensor.shape[0], input_tensor.shape[1]
    out = pl.pallas_call(
        _add_body,
        grid=(nh // HB, nb),
        in_specs=[
            pl.BlockSpec((1, HB, SEQ, SEQ), lambda hb, b: (b, hb, 0, 0)),
            pl.BlockSpec((HB, SEQ, SEQ), lambda hb, b: (hb, 0, 0)),
        ],
        out_specs=pl.BlockSpec((1, HB, SEQ, SEQ), lambda hb, b: (b, hb, 0, 0)),
        out_shape=jax.ShapeDtypeStruct(input_tensor.shape, input_tensor.dtype),
    )(input_tensor, bias3)
    return out
